# Initial kernel scaffold; baseline (speedup 1.0000x reference)
import functools

import jax
import jax.numpy as jnp
from jax import lax
from jax.experimental import pallas as pl
from jax.experimental.pallas import tpu as pltpu
from jax.experimental.pallas import tpu_sc as plsc

N_NODES = 100000
N_EDGES = 1600000
N_GRAPHS = 128
FEAT = 128
SLAB = 16

BN = 2500
N_BLOCKS = N_NODES // BN

NUM_SC = 2
NUM_TILES = 16
EDGES_PER_TILE = N_EDGES // NUM_TILES
CHUNK = 80
N_CHUNKS = EDGES_PER_TILE // CHUNK
ACC_ROWS_PER_TILE = N_NODES // NUM_TILES
ZROWS = ACC_ROWS_PER_TILE // 2


def _leaky(v):
    return jnp.where(v >= 0, v, 0.01 * v)


def _sc_pass(src_hbm, dst_hbm, table_hbm, out_hbm, tile, idx_s, idx_d, rows,
             zbuf, acc, sem):
    base = tile * ACC_ROWS_PER_TILE
    pltpu.sync_copy(zbuf, acc.at[pl.ds(base, ZROWS)])
    pltpu.sync_copy(zbuf, acc.at[pl.ds(base + ZROWS, ZROWS)])
    plsc.subcore_barrier()

    ebase = tile * EDGES_PER_TILE

    def step(i, _):
        off = ebase + i * CHUNK
        pltpu.sync_copy(src_hbm.at[pl.ds(off, CHUNK)], idx_s)
        pltpu.sync_copy(dst_hbm.at[pl.ds(off, CHUNK)], idx_d)
        pltpu.async_copy(table_hbm.at[idx_s], rows, sem).wait()
        pltpu.sync_copy(rows, acc.at[idx_d], add=True)
        return ()

    lax.fori_loop(0, N_CHUNKS, step, (), unroll=False)
    plsc.subcore_barrier()
    pltpu.sync_copy(acc.at[pl.ds(base, ACC_ROWS_PER_TILE)],
                    out_hbm.at[pl.ds(base, ACC_ROWS_PER_TILE)])


def _make_seg_sum(n_slabs):
    mesh = plsc.VectorSubcoreMesh(core_axis_name="c", subcore_axis_name="s")

    out_type = tuple(
        jax.ShapeDtypeStruct((N_NODES, SLAB), jnp.float32)
        for _ in range(n_slabs))

    @functools.partial(
        pl.kernel,
        out_type=out_type,
        mesh=mesh,
        scratch_types=dict(
            idx_s=pltpu.VMEM((CHUNK,), jnp.int32),
            idx_d=pltpu.VMEM((CHUNK,), jnp.int32),
            rows=pltpu.VMEM((CHUNK, SLAB), jnp.float32),
            zbuf=pltpu.VMEM((ZROWS, SLAB), jnp.float32),
            acc=pltpu.VMEM_SHARED((N_NODES, SLAB), jnp.float32),
            sem=pltpu.SemaphoreType.DMA,
        ),
    )
    def seg_sum(src_hbm, dst_hbm, *refs, idx_s, idx_d, rows, zbuf, acc, sem):
        tables = refs[:n_slabs]
        outs = refs[n_slabs:]
        core = lax.axis_index("c")
        tile = lax.axis_index("s")

        def zstep(i, _):
            zbuf[i, :] = jnp.zeros((SLAB,), jnp.float32)
            return ()
        lax.fori_loop(0, ZROWS, zstep, (), unroll=False)

        passes = n_slabs // NUM_SC
        for p in range(passes):
            for c in range(NUM_SC):
                j = c * passes + p

                @pl.when(core == c)
                def _(j=j):
                    _sc_pass(src_hbm, dst_hbm, tables[j], outs[j], tile,
                             idx_s, idx_d, rows, zbuf, acc, sem)

    return seg_sum


_seg_sum_2 = _make_seg_sum(2)
_seg_sum_4 = _make_seg_sum(4)


def _t0_body(x_ref, W_ref, b_ref, o0_ref, o1_ref):
    m = jnp.dot(x_ref[...], W_ref[...].T,
                preferred_element_type=jnp.float32) + b_ref[...]
    o0_ref[...] = m[:, :SLAB]
    o1_ref[...] = m[:, SLAB:]


def _stage_emb(x, W_emb, b_emb):
    return pl.pallas_call(
        _t0_body,
        grid=(N_BLOCKS,),
        in_specs=[
            pl.BlockSpec((BN, FEAT), lambda i: (i, 0)),
            pl.BlockSpec((32, FEAT), lambda i: (0, 0)),
            pl.BlockSpec((1, 32), lambda i: (0, 0)),
        ],
        out_specs=[
            pl.BlockSpec((BN, SLAB), lambda i: (i, 0)),
            pl.BlockSpec((BN, SLAB), lambda i: (i, 0)),
        ],
        out_shape=[
            jax.ShapeDtypeStruct((N_NODES, SLAB), jnp.float32),
            jax.ShapeDtypeStruct((N_NODES, SLAB), jnp.float32),
        ],
    )(x, W_emb, b_emb.reshape(1, 32))


def _mix_body(n_in, n_out, relu, refs):
    k = 0
    aggs = refs[k:k + n_in]; k += n_in
    hs = refs[k:k + n_in]; k += n_in
    Wl_ref, bl_ref, Wr_ref = refs[k:k + 3]; k += 3
    outs = refs[k:]
    Wl = Wl_ref[...]
    Wr = Wr_ref[...]
    res = bl_ref[...]
    for j in range(n_in):
        sl = slice(j * SLAB, (j + 1) * SLAB)
        res = res + jnp.dot(aggs[j][...], Wl[:, sl].T,
                            preferred_element_type=jnp.float32)
        res = res + jnp.dot(hs[j][...], Wr[:, sl].T,
                            preferred_element_type=jnp.float32)
    if relu:
        res = _leaky(res)
    for j in range(n_out):
        outs[j][...] = res[:, j * SLAB:(j + 1) * SLAB]


def _stage_mix(aggs, hs, Wl, bl, Wr, relu):
    n_in = len(aggs)
    d_out = Wl.shape[0]
    n_out = d_out // SLAB
    body = functools.partial(_mix_body, n_in, n_out, relu)
    return pl.pallas_call(
        body,
        grid=(N_BLOCKS,),
        in_specs=(
            [pl.BlockSpec((BN, SLAB), lambda i: (i, 0))] * (2 * n_in)
            + [pl.BlockSpec(Wl.shape, lambda i: (0, 0)),
               pl.BlockSpec((1, d_out), lambda i: (0, 0)),
               pl.BlockSpec(Wr.shape, lambda i: (0, 0))]
        ),
        out_specs=[pl.BlockSpec((BN, SLAB), lambda i: (i, 0))] * n_out,
        out_shape=[jax.ShapeDtypeStruct((N_NODES, SLAB), jnp.float32)] * n_out,
    )(*aggs, *hs, Wl, bl.reshape(1, d_out), Wr)


def _t3_body(refs):
    (a0, a1, a2, a3, h0, h1, h2, h3, Wl_ref, bl_ref, Wr_ref, batch_ref,
     Wf_ref, bf_ref, out_ref, acc_ref) = refs
    i = pl.program_id(0)
    Wl = Wl_ref[...]
    Wr = Wr_ref[...]
    res = bl_ref[...]
    aggs = (a0, a1, a2, a3)
    hs = (h0, h1, h2, h3)
    for j in range(4):
        sl = slice(j * SLAB, (j + 1) * SLAB)
        res = res + jnp.dot(aggs[j][...], Wl[:, sl].T,
                            preferred_element_type=jnp.float32)
        res = res + jnp.dot(hs[j][...], Wr[:, sl].T,
                            preferred_element_type=jnp.float32)
    ones = jnp.ones((BN, 8), jnp.float32)
    h3e = jnp.concatenate([res, ones], axis=1)
    batch_row = batch_ref[0]
    gids = lax.broadcasted_iota(jnp.int32, (N_GRAPHS, BN), 0)
    onehot_t = (gids == batch_row).astype(jnp.float32)
    part = jnp.dot(onehot_t, h3e, preferred_element_type=jnp.float32)

    @pl.when(i == 0)
    def _():
        acc_ref[...] = jnp.zeros_like(acc_ref)

    acc_ref[...] += part

    @pl.when(i == N_BLOCKS - 1)
    def _():
        acc = acc_ref[...]
        sums = acc[:, :FEAT]
        counts = acc[:, FEAT:FEAT + 1]
        pooled = sums / jnp.maximum(counts, 1.0)
        out_ref[...] = jnp.dot(pooled, Wf_ref[...].T,
                               preferred_element_type=jnp.float32) + bf_ref[...]


def _stage_final(aggs, hs, Wl3, bl3, Wr3, batch3d, W_flat, b_flat):
    return pl.pallas_call(
        lambda *refs: _t3_body(refs),
        grid=(N_BLOCKS,),
        in_specs=(
            [pl.BlockSpec((BN, SLAB), lambda i: (i, 0))] * 8
            + [pl.BlockSpec((FEAT, 64), lambda i: (0, 0)),
               pl.BlockSpec((1, FEAT), lambda i: (0, 0)),
               pl.BlockSpec((FEAT, 64), lambda i: (0, 0)),
               pl.BlockSpec((1, 1, BN), lambda i: (i, 0, 0)),
               pl.BlockSpec((256, FEAT), lambda i: (0, 0)),
               pl.BlockSpec((1, 256), lambda i: (0, 0))]
        ),
        out_specs=pl.BlockSpec((N_GRAPHS, 256), lambda i: (0, 0)),
        out_shape=jax.ShapeDtypeStruct((N_GRAPHS, 256), jnp.float32),
        scratch_shapes=[pltpu.VMEM((N_GRAPHS, FEAT + 8), jnp.float32)],
    )(*aggs, *hs, Wl3, bl3.reshape(1, FEAT), Wr3, batch3d,
      W_flat, b_flat.reshape(1, 256))


def kernel(x, edge_index, batch, W_emb, b_emb, Wl1, bl1, Wr1, Wl2, bl2, Wr2,
           Wl3, bl3, Wr3, W_flat, b_flat):
    src = edge_index[0]
    dst = edge_index[1]
    batch3d = batch.reshape(N_BLOCKS, 1, BN)

    h0 = _stage_emb(x, W_emb, b_emb)
    a1 = _seg_sum_2(src, dst, *h0)
    h1 = _stage_mix(a1, h0, Wl1, bl1, Wr1, relu=True)
    a2 = _seg_sum_2(src, dst, *h1)
    h2 = _stage_mix(a2, h1, Wl2, bl2, Wr2, relu=True)
    a3 = _seg_sum_4(src, dst, *h2)
    return _stage_final(a3, h2, Wl3, bl3, Wr3, batch3d, W_flat, b_flat)


# speedup vs baseline: 2.3065x; 2.3065x over previous
import functools

import jax
import jax.numpy as jnp
from jax import lax
from jax.experimental import pallas as pl
from jax.experimental.pallas import tpu as pltpu
from jax.experimental.pallas import tpu_sc as plsc

N_NODES = 100000
N_EDGES = 1600000
N_GRAPHS = 128
FEAT = 128
SLAB = 16

BN = 2000
N_BLOCKS = N_NODES // BN

NUM_SC = 2
NUM_TILES = 16
EDGES_PER_TILE = N_EDGES // NUM_TILES
CHUNK = 80
N_CHUNKS = EDGES_PER_TILE // CHUNK
ACC_ROWS_PER_TILE = 6256
ACC_PAD_ROWS = ACC_ROWS_PER_TILE * NUM_TILES
LAST_TILE_ROWS = N_NODES - 15 * ACC_ROWS_PER_TILE
ZROWS = 368
ZCOPIES = ACC_ROWS_PER_TILE // ZROWS


def _leaky(v):
    return jnp.where(v >= 0, v, 0.01 * v)


def _sc_pass(src_hbm, dst_hbm, table_hbm, out_hbm, tile, idx_s, idx_d, rows,
             zbuf, acc, sem):
    base = tile * ACC_ROWS_PER_TILE

    def zc(k, _):
        pltpu.sync_copy(zbuf, acc.at[pl.ds(base + k * ZROWS, ZROWS)])
        return ()

    lax.fori_loop(0, ZCOPIES, zc, (), unroll=False)
    plsc.subcore_barrier()

    ebase = tile * EDGES_PER_TILE

    def step(i, _):
        off = ebase + i * CHUNK
        pltpu.sync_copy(src_hbm.at[pl.ds(off, CHUNK)], idx_s)
        pltpu.sync_copy(dst_hbm.at[pl.ds(off, CHUNK)], idx_d)
        pltpu.async_copy(table_hbm.at[idx_s], rows, sem).wait()
        pltpu.sync_copy(rows, acc.at[idx_d], add=True)
        return ()

    lax.fori_loop(0, N_CHUNKS, step, (), unroll=False)
    plsc.subcore_barrier()

    @pl.when(tile < NUM_TILES - 1)
    def _():
        pltpu.sync_copy(acc.at[pl.ds(base, ACC_ROWS_PER_TILE)],
                        out_hbm.at[pl.ds(base, ACC_ROWS_PER_TILE)])

    @pl.when(tile == NUM_TILES - 1)
    def _():
        last = (NUM_TILES - 1) * ACC_ROWS_PER_TILE
        pltpu.sync_copy(acc.at[pl.ds(last, LAST_TILE_ROWS)],
                        out_hbm.at[pl.ds(last, LAST_TILE_ROWS)])


def _make_seg_sum(n_slabs):
    mesh = plsc.VectorSubcoreMesh(core_axis_name="c", subcore_axis_name="s")

    out_type = tuple(
        jax.ShapeDtypeStruct((N_NODES, SLAB), jnp.float32)
        for _ in range(n_slabs))

    @functools.partial(
        pl.kernel,
        out_type=out_type,
        mesh=mesh,
        scratch_types=dict(
            idx_s=pltpu.VMEM((CHUNK,), jnp.int32),
            idx_d=pltpu.VMEM((CHUNK,), jnp.int32),
            rows=pltpu.VMEM((CHUNK, SLAB), jnp.float32),
            zbuf=pltpu.VMEM((ZROWS, SLAB), jnp.float32),
            acc=pltpu.VMEM_SHARED((ACC_PAD_ROWS, SLAB), jnp.float32),
            sem=pltpu.SemaphoreType.DMA,
        ),
        compiler_params=pltpu.CompilerParams(use_tc_tiling_on_sc=False),
    )
    def seg_sum(src_hbm, dst_hbm, *refs, idx_s, idx_d, rows, zbuf, acc, sem):
        tables = refs[:n_slabs]
        outs = refs[n_slabs:]
        core = lax.axis_index("c")
        tile = lax.axis_index("s")

        def zstep(i, _):
            zbuf[i, :] = jnp.zeros((SLAB,), jnp.float32)
            return ()
        lax.fori_loop(0, ZROWS, zstep, (), unroll=False)

        passes = n_slabs // NUM_SC
        for p in range(passes):
            for c in range(NUM_SC):
                j = c * passes + p

                @pl.when(core == c)
                def _(j=j):
                    _sc_pass(src_hbm, dst_hbm, tables[j], outs[j], tile,
                             idx_s, idx_d, rows, zbuf, acc, sem)

    return seg_sum


_seg_sum_2 = _make_seg_sum(2)
_seg_sum_4 = _make_seg_sum(4)


def _t0_body(x_ref, W_ref, b_ref, o0_ref, o1_ref):
    m = jnp.dot(x_ref[...], W_ref[...].T,
                preferred_element_type=jnp.float32) + b_ref[...]
    o0_ref[...] = m[:, :SLAB]
    o1_ref[...] = m[:, SLAB:]


def _stage_emb(x, W_emb, b_emb):
    return pl.pallas_call(
        _t0_body,
        grid=(N_BLOCKS,),
        in_specs=[
            pl.BlockSpec((BN, FEAT), lambda i: (i, 0)),
            pl.BlockSpec((32, FEAT), lambda i: (0, 0)),
            pl.BlockSpec((1, 32), lambda i: (0, 0)),
        ],
        out_specs=[
            pl.BlockSpec((BN, SLAB), lambda i: (i, 0)),
            pl.BlockSpec((BN, SLAB), lambda i: (i, 0)),
        ],
        out_shape=[
            jax.ShapeDtypeStruct((N_NODES, SLAB), jnp.float32),
            jax.ShapeDtypeStruct((N_NODES, SLAB), jnp.float32),
        ],
    )(x, W_emb, b_emb.reshape(1, 32))


def _mix_body(n_in, n_out, relu, *refs):
    k = 0
    aggs = refs[k:k + n_in]; k += n_in
    hs = refs[k:k + n_in]; k += n_in
    Wl_ref, bl_ref, Wr_ref = refs[k:k + 3]; k += 3
    outs = refs[k:]
    Wl = Wl_ref[...]
    Wr = Wr_ref[...]
    res = bl_ref[...]
    for j in range(n_in):
        sl = slice(j * SLAB, (j + 1) * SLAB)
        res = res + jnp.dot(aggs[j][...], Wl[:, sl].T,
                            preferred_element_type=jnp.float32)
        res = res + jnp.dot(hs[j][...], Wr[:, sl].T,
                            preferred_element_type=jnp.float32)
    if relu:
        res = _leaky(res)
    for j in range(n_out):
        outs[j][...] = res[:, j * SLAB:(j + 1) * SLAB]


def _stage_mix(aggs, hs, Wl, bl, Wr, relu):
    n_in = len(aggs)
    d_out = Wl.shape[0]
    n_out = d_out // SLAB
    body = functools.partial(_mix_body, n_in, n_out, relu)
    return pl.pallas_call(
        body,
        grid=(N_BLOCKS,),
        in_specs=(
            [pl.BlockSpec((BN, SLAB), lambda i: (i, 0))] * (2 * n_in)
            + [pl.BlockSpec(Wl.shape, lambda i: (0, 0)),
               pl.BlockSpec((1, d_out), lambda i: (0, 0)),
               pl.BlockSpec(Wr.shape, lambda i: (0, 0))]
        ),
        out_specs=[pl.BlockSpec((BN, SLAB), lambda i: (i, 0))] * n_out,
        out_shape=[jax.ShapeDtypeStruct((N_NODES, SLAB), jnp.float32)] * n_out,
    )(*aggs, *hs, Wl, bl.reshape(1, d_out), Wr)


def _t3_body(refs):
    (a0, a1, a2, a3, h0, h1, h2, h3, Wl_ref, bl_ref, Wr_ref, batch_ref,
     Wf_ref, bf_ref, out_ref, acc_ref) = refs
    i = pl.program_id(0)
    Wl = Wl_ref[...]
    Wr = Wr_ref[...]
    res = bl_ref[...]
    aggs = (a0, a1, a2, a3)
    hs = (h0, h1, h2, h3)
    for j in range(4):
        sl = slice(j * SLAB, (j + 1) * SLAB)
        res = res + jnp.dot(aggs[j][...], Wl[:, sl].T,
                            preferred_element_type=jnp.float32)
        res = res + jnp.dot(hs[j][...], Wr[:, sl].T,
                            preferred_element_type=jnp.float32)
    ones = jnp.ones((BN, 8), jnp.float32)
    h3e = jnp.concatenate([res, ones], axis=1)
    batch_row = batch_ref[0]
    gids = lax.broadcasted_iota(jnp.int32, (N_GRAPHS, BN), 0)
    onehot_t = (gids == batch_row).astype(jnp.float32)
    part = jnp.dot(onehot_t, h3e, preferred_element_type=jnp.float32)

    @pl.when(i == 0)
    def _():
        acc_ref[...] = jnp.zeros_like(acc_ref)

    acc_ref[...] += part

    @pl.when(i == N_BLOCKS - 1)
    def _():
        acc = acc_ref[...]
        sums = acc[:, :FEAT]
        counts = acc[:, FEAT:FEAT + 1]
        pooled = sums / jnp.maximum(counts, 1.0)
        out_ref[...] = jnp.dot(pooled, Wf_ref[...].T,
                               preferred_element_type=jnp.float32) + bf_ref[...]


def _stage_final(aggs, hs, Wl3, bl3, Wr3, batch3d, W_flat, b_flat):
    return pl.pallas_call(
        lambda *refs: _t3_body(refs),
        grid=(N_BLOCKS,),
        in_specs=(
            [pl.BlockSpec((BN, SLAB), lambda i: (i, 0))] * 8
            + [pl.BlockSpec((FEAT, 64), lambda i: (0, 0)),
               pl.BlockSpec((1, FEAT), lambda i: (0, 0)),
               pl.BlockSpec((FEAT, 64), lambda i: (0, 0)),
               pl.BlockSpec((1, 1, BN), lambda i: (i, 0, 0)),
               pl.BlockSpec((256, FEAT), lambda i: (0, 0)),
               pl.BlockSpec((1, 256), lambda i: (0, 0))]
        ),
        out_specs=pl.BlockSpec((N_GRAPHS, 256), lambda i: (0, 0)),
        out_shape=jax.ShapeDtypeStruct((N_GRAPHS, 256), jnp.float32),
        scratch_shapes=[pltpu.VMEM((N_GRAPHS, FEAT + 8), jnp.float32)],
    )(*aggs, *hs, Wl3, bl3.reshape(1, FEAT), Wr3, batch3d,
      W_flat, b_flat.reshape(1, 256))


def kernel(x, edge_index, batch, W_emb, b_emb, Wl1, bl1, Wr1, Wl2, bl2, Wr2,
           Wl3, bl3, Wr3, W_flat, b_flat):
    src = edge_index[0]
    dst = edge_index[1]
    batch3d = batch.reshape(N_BLOCKS, 1, BN)

    h0 = _stage_emb(x, W_emb, b_emb)
    a1 = _seg_sum_2(src, dst, *h0)
    h1 = _stage_mix(a1, h0, Wl1, bl1, Wr1, relu=True)
    a2 = _seg_sum_2(src, dst, *h1)
    h2 = _stage_mix(a2, h1, Wl2, bl2, Wr2, relu=True)
    a3 = _seg_sum_4(src, dst, *h2)
    return _stage_final(a3, h2, Wl3, bl3, Wr3, batch3d, W_flat, b_flat)


# speedup vs baseline: 8.7427x; 3.7904x over previous
import functools

import jax
import jax.numpy as jnp
from jax import lax
from jax.experimental import pallas as pl
from jax.experimental.pallas import tpu as pltpu
from jax.experimental.pallas import tpu_sc as plsc

N_NODES = 100000
N_EDGES = 1600000
N_GRAPHS = 128
FEAT = 128
SLAB = 16

BN = 2000
N_BLOCKS = N_NODES // BN

NUM_SC = 2
NUM_TILES = 16
EDGES_PER_TILE = N_EDGES // NUM_TILES
CHUNK = 80
SUP = 10
N_SUP = EDGES_PER_TILE // (CHUNK * SUP)
ACC_ROWS_PER_TILE = 6256
ACC_PAD_ROWS = ACC_ROWS_PER_TILE * NUM_TILES
LAST_TILE_ROWS = N_NODES - 15 * ACC_ROWS_PER_TILE
ZROWS = 368
ZCOPIES = ACC_ROWS_PER_TILE // ZROWS


def _leaky(v):
    return jnp.where(v >= 0, v, 0.01 * v)


def _sc_pass(src_hbm, dst_hbm, table_hbm, out_hbm, tile, idx_s, idx_d, rows,
             zbuf, acc, gsem, ssem):
    base = tile * ACC_ROWS_PER_TILE

    def zc(k, _):
        pltpu.sync_copy(zbuf, acc.at[pl.ds(base + k * ZROWS, ZROWS)])
        return ()

    lax.fori_loop(0, ZCOPIES, zc, (), unroll=False)
    plsc.subcore_barrier()

    rbase = tile * (EDGES_PER_TILE // CHUNK)

    def step(s, _):
        roff = rbase + s * SUP
        pltpu.sync_copy(src_hbm.at[pl.ds(roff, SUP)], idx_s)
        pltpu.sync_copy(dst_hbm.at[pl.ds(roff, SUP)], idx_d)
        gds = [pltpu.async_copy(table_hbm.at[idx_s.at[j]], rows.at[j],
                                gsem.at[j])
               for j in range(SUP)]
        sds = []
        for j in range(SUP):
            gds[j].wait()
            sds.append(pltpu.async_copy(rows.at[j], acc.at[idx_d.at[j]],
                                        ssem.at[j], add=True))
        for j in range(SUP):
            sds[j].wait()
        return ()

    lax.fori_loop(0, N_SUP, step, (), unroll=False)
    plsc.subcore_barrier()

    @pl.when(tile < NUM_TILES - 1)
    def _():
        pltpu.sync_copy(acc.at[pl.ds(base, ACC_ROWS_PER_TILE)],
                        out_hbm.at[pl.ds(base, ACC_ROWS_PER_TILE)])

    @pl.when(tile == NUM_TILES - 1)
    def _():
        last = (NUM_TILES - 1) * ACC_ROWS_PER_TILE
        pltpu.sync_copy(acc.at[pl.ds(last, LAST_TILE_ROWS)],
                        out_hbm.at[pl.ds(last, LAST_TILE_ROWS)])


def _make_seg_sum(n_slabs):
    mesh = plsc.VectorSubcoreMesh(core_axis_name="c", subcore_axis_name="s")

    out_type = tuple(
        jax.ShapeDtypeStruct((N_NODES, SLAB), jnp.float32)
        for _ in range(n_slabs))

    @functools.partial(
        pl.kernel,
        out_type=out_type,
        mesh=mesh,
        scratch_types=dict(
            idx_s=pltpu.VMEM((SUP, CHUNK), jnp.int32),
            idx_d=pltpu.VMEM((SUP, CHUNK), jnp.int32),
            rows=pltpu.VMEM((SUP, CHUNK, SLAB), jnp.float32),
            zbuf=pltpu.VMEM((ZROWS, SLAB), jnp.float32),
            acc=pltpu.VMEM_SHARED((ACC_PAD_ROWS, SLAB), jnp.float32),
            gsem=pltpu.SemaphoreType.DMA((SUP,)),
            ssem=pltpu.SemaphoreType.DMA((SUP,)),
        ),
        compiler_params=pltpu.CompilerParams(use_tc_tiling_on_sc=False),
    )
    def seg_sum(src_hbm, dst_hbm, *refs, idx_s, idx_d, rows, zbuf, acc,
                gsem, ssem):
        tables = refs[:n_slabs]
        outs = refs[n_slabs:]
        core = lax.axis_index("c")
        tile = lax.axis_index("s")

        def zstep(i, _):
            zbuf[i, :] = jnp.zeros((SLAB,), jnp.float32)
            return ()
        lax.fori_loop(0, ZROWS, zstep, (), unroll=False)

        passes = n_slabs // NUM_SC
        for p in range(passes):
            for c in range(NUM_SC):
                j = c * passes + p

                @pl.when(core == c)
                def _(j=j):
                    _sc_pass(src_hbm, dst_hbm, tables[j], outs[j], tile,
                             idx_s, idx_d, rows, zbuf, acc, gsem, ssem)

    return seg_sum


_seg_sum_2 = _make_seg_sum(2)
_seg_sum_4 = _make_seg_sum(4)


def _t0_body(x_ref, W_ref, b_ref, o0_ref, o1_ref):
    m = jnp.dot(x_ref[...], W_ref[...].T,
                preferred_element_type=jnp.float32) + b_ref[...]
    o0_ref[...] = m[:, :SLAB]
    o1_ref[...] = m[:, SLAB:]


def _stage_emb(x, W_emb, b_emb):
    return pl.pallas_call(
        _t0_body,
        grid=(N_BLOCKS,),
        in_specs=[
            pl.BlockSpec((BN, FEAT), lambda i: (i, 0)),
            pl.BlockSpec((32, FEAT), lambda i: (0, 0)),
            pl.BlockSpec((1, 32), lambda i: (0, 0)),
        ],
        out_specs=[
            pl.BlockSpec((BN, SLAB), lambda i: (i, 0)),
            pl.BlockSpec((BN, SLAB), lambda i: (i, 0)),
        ],
        out_shape=[
            jax.ShapeDtypeStruct((N_NODES, SLAB), jnp.float32),
            jax.ShapeDtypeStruct((N_NODES, SLAB), jnp.float32),
        ],
    )(x, W_emb, b_emb.reshape(1, 32))


def _mix_body(n_in, n_out, relu, *refs):
    k = 0
    aggs = refs[k:k + n_in]; k += n_in
    hs = refs[k:k + n_in]; k += n_in
    Wl_ref, bl_ref, Wr_ref = refs[k:k + 3]; k += 3
    outs = refs[k:]
    Wl = Wl_ref[...]
    Wr = Wr_ref[...]
    res = bl_ref[...]
    for j in range(n_in):
        sl = slice(j * SLAB, (j + 1) * SLAB)
        res = res + jnp.dot(aggs[j][...], Wl[:, sl].T,
                            preferred_element_type=jnp.float32)
        res = res + jnp.dot(hs[j][...], Wr[:, sl].T,
                            preferred_element_type=jnp.float32)
    if relu:
        res = _leaky(res)
    for j in range(n_out):
        outs[j][...] = res[:, j * SLAB:(j + 1) * SLAB]


def _stage_mix(aggs, hs, Wl, bl, Wr, relu):
    n_in = len(aggs)
    d_out = Wl.shape[0]
    n_out = d_out // SLAB
    body = functools.partial(_mix_body, n_in, n_out, relu)
    return pl.pallas_call(
        body,
        grid=(N_BLOCKS,),
        in_specs=(
            [pl.BlockSpec((BN, SLAB), lambda i: (i, 0))] * (2 * n_in)
            + [pl.BlockSpec(Wl.shape, lambda i: (0, 0)),
               pl.BlockSpec((1, d_out), lambda i: (0, 0)),
               pl.BlockSpec(Wr.shape, lambda i: (0, 0))]
        ),
        out_specs=[pl.BlockSpec((BN, SLAB), lambda i: (i, 0))] * n_out,
        out_shape=[jax.ShapeDtypeStruct((N_NODES, SLAB), jnp.float32)] * n_out,
    )(*aggs, *hs, Wl, bl.reshape(1, d_out), Wr)


def _t3_body(refs):
    (a0, a1, a2, a3, h0, h1, h2, h3, Wl_ref, bl_ref, Wr_ref, batch_ref,
     Wf_ref, bf_ref, out_ref, acc_ref) = refs
    i = pl.program_id(0)
    Wl = Wl_ref[...]
    Wr = Wr_ref[...]
    res = bl_ref[...]
    aggs = (a0, a1, a2, a3)
    hs = (h0, h1, h2, h3)
    for j in range(4):
        sl = slice(j * SLAB, (j + 1) * SLAB)
        res = res + jnp.dot(aggs[j][...], Wl[:, sl].T,
                            preferred_element_type=jnp.float32)
        res = res + jnp.dot(hs[j][...], Wr[:, sl].T,
                            preferred_element_type=jnp.float32)
    ones = jnp.ones((BN, 8), jnp.float32)
    h3e = jnp.concatenate([res, ones], axis=1)
    batch_row = batch_ref[0]
    gids = lax.broadcasted_iota(jnp.int32, (N_GRAPHS, BN), 0)
    onehot_t = (gids == batch_row).astype(jnp.float32)
    part = jnp.dot(onehot_t, h3e, preferred_element_type=jnp.float32)

    @pl.when(i == 0)
    def _():
        acc_ref[...] = jnp.zeros_like(acc_ref)

    acc_ref[...] += part

    @pl.when(i == N_BLOCKS - 1)
    def _():
        acc = acc_ref[...]
        sums = acc[:, :FEAT]
        counts = acc[:, FEAT:FEAT + 1]
        pooled = sums / jnp.maximum(counts, 1.0)
        out_ref[...] = jnp.dot(pooled, Wf_ref[...].T,
                               preferred_element_type=jnp.float32) + bf_ref[...]


def _stage_final(aggs, hs, Wl3, bl3, Wr3, batch3d, W_flat, b_flat):
    return pl.pallas_call(
        lambda *refs: _t3_body(refs),
        grid=(N_BLOCKS,),
        in_specs=(
            [pl.BlockSpec((BN, SLAB), lambda i: (i, 0))] * 8
            + [pl.BlockSpec((FEAT, 64), lambda i: (0, 0)),
               pl.BlockSpec((1, FEAT), lambda i: (0, 0)),
               pl.BlockSpec((FEAT, 64), lambda i: (0, 0)),
               pl.BlockSpec((1, 1, BN), lambda i: (i, 0, 0)),
               pl.BlockSpec((256, FEAT), lambda i: (0, 0)),
               pl.BlockSpec((1, 256), lambda i: (0, 0))]
        ),
        out_specs=pl.BlockSpec((N_GRAPHS, 256), lambda i: (0, 0)),
        out_shape=jax.ShapeDtypeStruct((N_GRAPHS, 256), jnp.float32),
        scratch_shapes=[pltpu.VMEM((N_GRAPHS, FEAT + 8), jnp.float32)],
    )(*aggs, *hs, Wl3, bl3.reshape(1, FEAT), Wr3, batch3d,
      W_flat, b_flat.reshape(1, 256))


def kernel(x, edge_index, batch, W_emb, b_emb, Wl1, bl1, Wr1, Wl2, bl2, Wr2,
           Wl3, bl3, Wr3, W_flat, b_flat):
    src = edge_index[0].reshape(-1, CHUNK)
    dst = edge_index[1].reshape(-1, CHUNK)
    batch3d = batch.reshape(N_BLOCKS, 1, BN)

    h0 = _stage_emb(x, W_emb, b_emb)
    a1 = _seg_sum_2(src, dst, *h0)
    h1 = _stage_mix(a1, h0, Wl1, bl1, Wr1, relu=True)
    a2 = _seg_sum_2(src, dst, *h1)
    h2 = _stage_mix(a2, h1, Wl2, bl2, Wr2, relu=True)
    a3 = _seg_sum_4(src, dst, *h2)
    return _stage_final(a3, h2, Wl3, bl3, Wr3, batch3d, W_flat, b_flat)


# speedup vs baseline: 11.7719x; 1.3465x over previous
import functools

import jax
import jax.numpy as jnp
from jax import lax
from jax.experimental import pallas as pl
from jax.experimental.pallas import tpu as pltpu
from jax.experimental.pallas import tpu_sc as plsc

N_NODES = 100000
N_EDGES = 1600000
N_GRAPHS = 128
FEAT = 128
SLAB = 16

NP = 102400
PR = NP * SLAB // 128
PB = 256
N_BLOCKS = PR // PB
XP_BLOCKS = 49

NUM_SC = 2
NUM_TILES = 16
EDGES_PER_TILE = N_EDGES // NUM_TILES
CHUNK = 80
SUP = 10
N_SUP = EDGES_PER_TILE // (CHUNK * SUP)
ACC_ROWS_PER_TILE = NP // NUM_TILES
ZROWS = 200
ZCOPIES = ACC_ROWS_PER_TILE // ZROWS


def _leaky(v):
    return jnp.where(v >= 0, v, 0.01 * v)


def _kron8(m):
    return jnp.kron(jnp.eye(8, dtype=jnp.float32), m)


def _pack_bias(b):
    return jnp.tile(b.reshape(-1, SLAB), (1, 8)).reshape(-1, 8 * SLAB)


def _bd_weights(W):
    n_out = W.shape[0] // SLAB
    n_in = W.shape[1] // SLAB
    return jnp.stack([
        jnp.stack([
            _kron8(W[o * SLAB:(o + 1) * SLAB, i * SLAB:(i + 1) * SLAB].T)
            for o in range(n_out)])
        for i in range(n_in)])


def _sc_pass(src_hbm, dst_hbm, table_hbm, out_hbm, tile, idx_s, idx_d, rows,
             zbuf, acc, gsem, ssem):
    base = tile * ACC_ROWS_PER_TILE

    def zc(k, _):
        pltpu.sync_copy(zbuf, acc.at[pl.ds(base + k * ZROWS, ZROWS)])
        return ()

    lax.fori_loop(0, ZCOPIES, zc, (), unroll=False)
    plsc.subcore_barrier()

    rbase = tile * (EDGES_PER_TILE // CHUNK)

    def step(s, _):
        roff = rbase + s * SUP
        pltpu.sync_copy(src_hbm.at[pl.ds(roff, SUP)], idx_s)
        pltpu.sync_copy(dst_hbm.at[pl.ds(roff, SUP)], idx_d)
        gds = [pltpu.async_copy(table_hbm.at[idx_s.at[j]], rows.at[j],
                                gsem.at[j])
               for j in range(SUP)]
        sds = []
        for j in range(SUP):
            gds[j].wait()
            sds.append(pltpu.async_copy(rows.at[j], acc.at[idx_d.at[j]],
                                        ssem.at[j], add=True))
        for j in range(SUP):
            sds[j].wait()
        return ()

    lax.fori_loop(0, N_SUP, step, (), unroll=False)
    plsc.subcore_barrier()
    pltpu.sync_copy(acc.at[pl.ds(base, ACC_ROWS_PER_TILE)],
                    out_hbm.at[pl.ds(base, ACC_ROWS_PER_TILE)])


def _make_seg_sum(n_slabs):
    mesh = plsc.VectorSubcoreMesh(core_axis_name="c", subcore_axis_name="s")

    out_type = tuple(
        jax.ShapeDtypeStruct((NP, SLAB), jnp.float32)
        for _ in range(n_slabs))

    @functools.partial(
        pl.kernel,
        out_type=out_type,
        mesh=mesh,
        scratch_types=dict(
            idx_s=pltpu.VMEM((SUP, CHUNK), jnp.int32),
            idx_d=pltpu.VMEM((SUP, CHUNK), jnp.int32),
            rows=pltpu.VMEM((SUP, CHUNK, SLAB), jnp.float32),
            zbuf=pltpu.VMEM((ZROWS, SLAB), jnp.float32),
            acc=pltpu.VMEM_SHARED((NP, SLAB), jnp.float32),
            gsem=pltpu.SemaphoreType.DMA((SUP,)),
            ssem=pltpu.SemaphoreType.DMA((SUP,)),
        ),
        compiler_params=pltpu.CompilerParams(use_tc_tiling_on_sc=False),
    )
    def seg_sum(src_hbm, dst_hbm, *refs, idx_s, idx_d, rows, zbuf, acc,
                gsem, ssem):
        tables = refs[:n_slabs]
        outs = refs[n_slabs:]
        core = lax.axis_index("c")
        tile = lax.axis_index("s")

        def zstep(i, _):
            zbuf[i, :] = jnp.zeros((SLAB,), jnp.float32)
            return ()
        lax.fori_loop(0, ZROWS, zstep, (), unroll=False)

        passes = n_slabs // NUM_SC
        for p in range(passes):
            for c in range(NUM_SC):
                j = c * passes + p

                @pl.when(core == c)
                def _(j=j):
                    _sc_pass(src_hbm, dst_hbm, tables[j], outs[j], tile,
                             idx_s, idx_d, rows, zbuf, acc, gsem, ssem)

    return seg_sum


_seg_sum_2 = _make_seg_sum(2)
_seg_sum_4 = _make_seg_sum(4)


def _seg(seg_fn, src, dst, slabs):
    outs = seg_fn(src, dst, *[s.reshape(NP, SLAB) for s in slabs])
    return [o.reshape(PR, 128) for o in outs]


def _t0_body(xp_ref, g_ref, b_ref, o0_ref, o1_ref):
    i = pl.program_id(0)
    rows = (lax.broadcasted_iota(jnp.int32, (PB, 8 * FEAT), 0)
            + jnp.minimum(i, XP_BLOCKS - 1) * PB)
    xp = jnp.where(rows < N_NODES * SLAB // 128, xp_ref[...], 0.0)
    for s, o_ref in enumerate((o0_ref, o1_ref)):
        o_ref[...] = jnp.dot(xp, g_ref[s], preferred_element_type=jnp.float32
                             ) + b_ref[s:s + 1, :]


def _stage_emb(xp, G, bpk):
    return pl.pallas_call(
        _t0_body,
        grid=(N_BLOCKS,),
        in_specs=[
            pl.BlockSpec((PB, 1024),
                         lambda i: (jnp.minimum(i, XP_BLOCKS - 1), 0)),
            pl.BlockSpec((2, 1024, 128), lambda i: (0, 0, 0)),
            pl.BlockSpec((2, 128), lambda i: (0, 0)),
        ],
        out_specs=[
            pl.BlockSpec((PB, 128), lambda i: (i, 0)),
            pl.BlockSpec((PB, 128), lambda i: (i, 0)),
        ],
        out_shape=[
            jax.ShapeDtypeStruct((PR, 128), jnp.float32),
            jax.ShapeDtypeStruct((PR, 128), jnp.float32),
        ],
    )(xp, G, bpk)


def _mix_body(n_in, n_out, relu, *refs):
    aggs = refs[:n_in]
    hs = refs[n_in:2 * n_in]
    BDL_ref, BDR_ref, b_ref = refs[2 * n_in:2 * n_in + 3]
    outs = refs[2 * n_in + 3:]
    for o in range(n_out):
        res = jnp.zeros((PB, 128), jnp.float32) + b_ref[o:o + 1, :]
        for i in range(n_in):
            res = res + jnp.dot(aggs[i][...], BDL_ref[i, o],
                                preferred_element_type=jnp.float32)
            res = res + jnp.dot(hs[i][...], BDR_ref[i, o],
                                preferred_element_type=jnp.float32)
        if relu:
            res = _leaky(res)
        outs[o][...] = res


def _stage_mix(aggs, hs, BDL, BDR, bpk, relu):
    n_in, n_out = BDL.shape[0], BDL.shape[1]
    body = functools.partial(_mix_body, n_in, n_out, relu)
    return pl.pallas_call(
        body,
        grid=(N_BLOCKS,),
        in_specs=(
            [pl.BlockSpec((PB, 128), lambda i: (i, 0))] * (2 * n_in)
            + [pl.BlockSpec(BDL.shape, lambda i: (0,) * 4),
               pl.BlockSpec(BDR.shape, lambda i: (0,) * 4),
               pl.BlockSpec(bpk.shape, lambda i: (0, 0))]
        ),
        out_specs=[pl.BlockSpec((PB, 128), lambda i: (i, 0))] * n_out,
        out_shape=[jax.ShapeDtypeStruct((PR, 128), jnp.float32)] * n_out,
    )(*aggs, *hs, BDL, BDR, bpk)


def _t3_body(refs):
    (a0, a1, a2, a3, h0, h1, h2, h3, Wl_ref, bl_ref, Wr_ref, bat_ref,
     Wf_ref, bf_ref, out_ref, acc_ref) = refs
    i = pl.program_id(0)
    Wl = Wl_ref[...]
    Wr = Wr_ref[...]
    aggs = (a0, a1, a2, a3)
    hs = (h0, h1, h2, h3)

    @pl.when(i == 0)
    def _():
        acc_ref[...] = jnp.zeros_like(acc_ref)

    part = jnp.zeros((N_GRAPHS, FEAT + 8), jnp.float32)
    ones = jnp.ones((PB, 8), jnp.float32)
    for b in range(8):
        sl = slice(SLAB * b, SLAB * (b + 1))
        A_b = jnp.concatenate([aggs[s][:, sl] for s in range(4)], axis=1)
        H_b = jnp.concatenate([hs[s][:, sl] for s in range(4)], axis=1)
        h3_b = (jnp.dot(A_b, Wl.T, preferred_element_type=jnp.float32)
                + bl_ref[...]
                + jnp.dot(H_b, Wr.T, preferred_element_type=jnp.float32))
        h3e_b = jnp.concatenate([h3_b, ones], axis=1)
        brow = bat_ref[0, b:b + 1, :]
        gids = lax.broadcasted_iota(jnp.int32, (N_GRAPHS, PB), 0)
        onehot_t = (gids == brow).astype(jnp.float32)
        part = part + jnp.dot(onehot_t, h3e_b,
                              preferred_element_type=jnp.float32)
    acc_ref[...] += part

    @pl.when(i == N_BLOCKS - 1)
    def _():
        acc = acc_ref[...]
        sums = acc[:, :FEAT]
        counts = acc[:, FEAT:FEAT + 1]
        pooled = sums / jnp.maximum(counts, 1.0)
        out_ref[...] = jnp.dot(pooled, Wf_ref[...].T,
                               preferred_element_type=jnp.float32) + bf_ref[...]


def _stage_final(aggs, hs, Wl3, bl3, Wr3, batch_lanes, W_flat, b_flat):
    return pl.pallas_call(
        lambda *refs: _t3_body(refs),
        grid=(N_BLOCKS,),
        in_specs=(
            [pl.BlockSpec((PB, 128), lambda i: (i, 0))] * 8
            + [pl.BlockSpec((FEAT, 64), lambda i: (0, 0)),
               pl.BlockSpec((1, FEAT), lambda i: (0, 0)),
               pl.BlockSpec((FEAT, 64), lambda i: (0, 0)),
               pl.BlockSpec((1, 8, PB), lambda i: (i, 0, 0)),
               pl.BlockSpec((256, FEAT), lambda i: (0, 0)),
               pl.BlockSpec((1, 256), lambda i: (0, 0))]
        ),
        out_specs=pl.BlockSpec((N_GRAPHS, 256), lambda i: (0, 0)),
        out_shape=jax.ShapeDtypeStruct((N_GRAPHS, 256), jnp.float32),
        scratch_shapes=[pltpu.VMEM((N_GRAPHS, FEAT + 8), jnp.float32)],
    )(*aggs, *hs, Wl3, bl3.reshape(1, FEAT), Wr3, batch_lanes,
      W_flat, b_flat.reshape(1, 256))


def kernel(x, edge_index, batch, W_emb, b_emb, Wl1, bl1, Wr1, Wl2, bl2, Wr2,
           Wl3, bl3, Wr3, W_flat, b_flat):
    src = edge_index[0].reshape(-1, CHUNK)
    dst = edge_index[1].reshape(-1, CHUNK)
    xp = x.reshape(-1, 8 * FEAT)

    batch_pad = jnp.concatenate(
        [batch, jnp.full((NP - N_NODES,), N_GRAPHS, batch.dtype)])
    batch_lanes = batch_pad.reshape(N_BLOCKS, PB, 8).transpose(0, 2, 1)

    G = jnp.stack([_kron8(W_emb[s * SLAB:(s + 1) * SLAB, :].T)
                   for s in range(2)])
    bpk0 = _pack_bias(b_emb)
    BDL1, BDR1, bpk1 = _bd_weights(Wl1), _bd_weights(Wr1), _pack_bias(bl1)
    BDL2, BDR2, bpk2 = _bd_weights(Wl2), _bd_weights(Wr2), _pack_bias(bl2)

    h0 = _stage_emb(xp, G, bpk0)
    a1 = _seg(_seg_sum_2, src, dst, h0)
    h1 = _stage_mix(a1, h0, BDL1, BDR1, bpk1, relu=True)
    a2 = _seg(_seg_sum_2, src, dst, h1)
    h2 = _stage_mix(a2, h1, BDL2, BDR2, bpk2, relu=True)
    a3 = _seg(_seg_sum_4, src, dst, h2)
    return _stage_final(a3, h2, Wl3, bl3, Wr3, batch_lanes, W_flat, b_flat)


# speedup vs baseline: 13.6518x; 1.1597x over previous
import functools

import jax
import jax.numpy as jnp
from jax import lax
from jax.experimental import pallas as pl
from jax.experimental.pallas import tpu as pltpu
from jax.experimental.pallas import tpu_sc as plsc

N_NODES = 100000
N_EDGES = 1600000
N_GRAPHS = 128
FEAT = 128
SLAB = 16

NP = 102400
PR = NP * SLAB // 128
PB = 256
N_BLOCKS = PR // PB
XP_BLOCKS = 49

NUM_SC = 2
NUM_TILES = 16
EDGES_PER_TILE = N_EDGES // NUM_TILES
CHUNK = 80
SUP = 10
N_SUP = EDGES_PER_TILE // (CHUNK * SUP)
ACC_ROWS_PER_TILE = NP // NUM_TILES
ZROWS = 200
ZCOPIES = ACC_ROWS_PER_TILE // ZROWS


def _leaky(v):
    return jnp.where(v >= 0, v, 0.01 * v)


def _kron8(m):
    return jnp.kron(jnp.eye(8, dtype=jnp.float32), m)


def _pack_bias(b):
    return jnp.tile(b.reshape(-1, SLAB), (1, 8)).reshape(-1, 8 * SLAB)


def _bd_weights(W):
    n_out = W.shape[0] // SLAB
    n_in = W.shape[1] // SLAB
    return jnp.stack([
        jnp.stack([
            _kron8(W[o * SLAB:(o + 1) * SLAB, i * SLAB:(i + 1) * SLAB].T)
            for o in range(n_out)])
        for i in range(n_in)])


def _sc_pass(src_hbm, dst_hbm, table_hbm, out_hbm, tile, idx_s, idx_d, rows,
             zbuf, acc, gsem, ssem, isem, dsem):
    base = tile * ACC_ROWS_PER_TILE

    def zc(k, _):
        pltpu.sync_copy(zbuf, acc.at[pl.ds(base + k * ZROWS, ZROWS)])
        return ()

    lax.fori_loop(0, ZCOPIES, zc, (), unroll=False)
    plsc.subcore_barrier()

    rbase = tile * (EDGES_PER_TILE // CHUNK)

    def step(s, _):
        roff = rbase + s * SUP
        d1 = pltpu.async_copy(src_hbm.at[pl.ds(roff, SUP)], idx_s, isem)
        d2 = pltpu.async_copy(dst_hbm.at[pl.ds(roff, SUP)], idx_d, dsem)
        d1.wait()
        d2.wait()
        gds = [pltpu.async_copy(table_hbm.at[idx_s.at[j]], rows.at[j],
                                gsem.at[j])
               for j in range(SUP)]
        sds = []
        for j in range(SUP):
            gds[j].wait()
            sds.append(pltpu.async_copy(rows.at[j], acc.at[idx_d.at[j]],
                                        ssem.at[j], add=True))
        for j in range(SUP):
            sds[j].wait()
        return ()

    lax.fori_loop(0, N_SUP, step, (), unroll=False)
    plsc.subcore_barrier()
    pltpu.sync_copy(acc.at[pl.ds(base, ACC_ROWS_PER_TILE)],
                    out_hbm.at[pl.ds(base, ACC_ROWS_PER_TILE)])


def _make_seg_sum(n_slabs):
    mesh = plsc.VectorSubcoreMesh(core_axis_name="c", subcore_axis_name="s")

    out_type = tuple(
        jax.ShapeDtypeStruct((NP, SLAB), jnp.float32)
        for _ in range(n_slabs))

    @functools.partial(
        pl.kernel,
        out_type=out_type,
        mesh=mesh,
        scratch_types=dict(
            idx_s=pltpu.VMEM((SUP, CHUNK), jnp.int32),
            idx_d=pltpu.VMEM((SUP, CHUNK), jnp.int32),
            rows=pltpu.VMEM((SUP, CHUNK, SLAB), jnp.float32),
            zbuf=pltpu.VMEM((ZROWS, SLAB), jnp.float32),
            acc=pltpu.VMEM_SHARED((NP, SLAB), jnp.float32),
            gsem=pltpu.SemaphoreType.DMA((SUP,)),
            ssem=pltpu.SemaphoreType.DMA((SUP,)),
            isem=pltpu.SemaphoreType.DMA,
            dsem=pltpu.SemaphoreType.DMA,
        ),
        compiler_params=pltpu.CompilerParams(use_tc_tiling_on_sc=False),
    )
    def seg_sum(src_hbm, dst_hbm, *refs, idx_s, idx_d, rows, zbuf, acc,
                gsem, ssem, isem, dsem):
        tables = refs[:n_slabs]
        outs = refs[n_slabs:]
        core = lax.axis_index("c")
        tile = lax.axis_index("s")

        def zstep(i, _):
            zbuf[i, :] = jnp.zeros((SLAB,), jnp.float32)
            return ()
        lax.fori_loop(0, ZROWS, zstep, (), unroll=False)

        passes = n_slabs // NUM_SC
        for p in range(passes):
            for c in range(NUM_SC):
                j = c * passes + p

                @pl.when(core == c)
                def _(j=j):
                    _sc_pass(src_hbm, dst_hbm, tables[j], outs[j], tile,
                             idx_s, idx_d, rows, zbuf, acc, gsem, ssem,
                             isem, dsem)

    return seg_sum


_seg_sum_2 = _make_seg_sum(2)
_seg_sum_4 = _make_seg_sum(4)


def _seg(seg_fn, src, dst, slabs):
    outs = seg_fn(src, dst, *[s.reshape(NP, SLAB) for s in slabs])
    return [o.reshape(PR, 128) for o in outs]


def _t0_body(xp_ref, g_ref, b_ref, o0_ref, o1_ref):
    i = pl.program_id(0)
    rows = (lax.broadcasted_iota(jnp.int32, (PB, 8 * FEAT), 0)
            + jnp.minimum(i, XP_BLOCKS - 1) * PB)
    xp = jnp.where(rows < N_NODES * SLAB // 128, xp_ref[...], 0.0)
    for s, o_ref in enumerate((o0_ref, o1_ref)):
        o_ref[...] = jnp.dot(xp, g_ref[s], preferred_element_type=jnp.float32
                             ) + b_ref[s:s + 1, :]


def _stage_emb(xp, G, bpk):
    return pl.pallas_call(
        _t0_body,
        grid=(N_BLOCKS,),
        in_specs=[
            pl.BlockSpec((PB, 1024),
                         lambda i: (jnp.minimum(i, XP_BLOCKS - 1), 0)),
            pl.BlockSpec((2, 1024, 128), lambda i: (0, 0, 0)),
            pl.BlockSpec((2, 128), lambda i: (0, 0)),
        ],
        out_specs=[
            pl.BlockSpec((PB, 128), lambda i: (i, 0)),
            pl.BlockSpec((PB, 128), lambda i: (i, 0)),
        ],
        out_shape=[
            jax.ShapeDtypeStruct((PR, 128), jnp.float32),
            jax.ShapeDtypeStruct((PR, 128), jnp.float32),
        ],
    )(xp, G, bpk)


def _mix_body(n_in, n_out, relu, *refs):
    aggs = refs[:n_in]
    hs = refs[n_in:2 * n_in]
    BDL_ref, BDR_ref, b_ref = refs[2 * n_in:2 * n_in + 3]
    outs = refs[2 * n_in + 3:]
    for o in range(n_out):
        res = jnp.zeros((PB, 128), jnp.float32) + b_ref[o:o + 1, :]
        for i in range(n_in):
            res = res + jnp.dot(aggs[i][...], BDL_ref[i, o],
                                preferred_element_type=jnp.float32)
            res = res + jnp.dot(hs[i][...], BDR_ref[i, o],
                                preferred_element_type=jnp.float32)
        if relu:
            res = _leaky(res)
        outs[o][...] = res


def _stage_mix(aggs, hs, BDL, BDR, bpk, relu):
    n_in, n_out = BDL.shape[0], BDL.shape[1]
    body = functools.partial(_mix_body, n_in, n_out, relu)
    return pl.pallas_call(
        body,
        grid=(N_BLOCKS,),
        in_specs=(
            [pl.BlockSpec((PB, 128), lambda i: (i, 0))] * (2 * n_in)
            + [pl.BlockSpec(BDL.shape, lambda i: (0,) * 4),
               pl.BlockSpec(BDR.shape, lambda i: (0,) * 4),
               pl.BlockSpec(bpk.shape, lambda i: (0, 0))]
        ),
        out_specs=[pl.BlockSpec((PB, 128), lambda i: (i, 0))] * n_out,
        out_shape=[jax.ShapeDtypeStruct((PR, 128), jnp.float32)] * n_out,
    )(*aggs, *hs, BDL, BDR, bpk)


def _t3_body(refs):
    (a0, a1, a2, a3, h0, h1, h2, h3, Wl_ref, bl_ref, Wr_ref, bat_ref,
     Wf_ref, bf_ref, out_ref, acc_ref) = refs
    i = pl.program_id(0)
    Wl = Wl_ref[...]
    Wr = Wr_ref[...]
    aggs = (a0, a1, a2, a3)
    hs = (h0, h1, h2, h3)

    @pl.when(i == 0)
    def _():
        acc_ref[...] = jnp.zeros_like(acc_ref)

    part = jnp.zeros((N_GRAPHS, FEAT + 8), jnp.float32)
    ones = jnp.ones((PB, 8), jnp.float32)
    for b in range(8):
        sl = slice(SLAB * b, SLAB * (b + 1))
        A_b = jnp.concatenate([aggs[s][:, sl] for s in range(4)], axis=1)
        H_b = jnp.concatenate([hs[s][:, sl] for s in range(4)], axis=1)
        h3_b = (jnp.dot(A_b, Wl.T, preferred_element_type=jnp.float32)
                + bl_ref[...]
                + jnp.dot(H_b, Wr.T, preferred_element_type=jnp.float32))
        h3e_b = jnp.concatenate([h3_b, ones], axis=1)
        brow = bat_ref[0, b:b + 1, :]
        gids = lax.broadcasted_iota(jnp.int32, (N_GRAPHS, PB), 0)
        onehot_t = (gids == brow).astype(jnp.float32)
        part = part + jnp.dot(onehot_t, h3e_b,
                              preferred_element_type=jnp.float32)
    acc_ref[...] += part

    @pl.when(i == N_BLOCKS - 1)
    def _():
        acc = acc_ref[...]
        sums = acc[:, :FEAT]
        counts = acc[:, FEAT:FEAT + 1]
        pooled = sums / jnp.maximum(counts, 1.0)
        out_ref[...] = jnp.dot(pooled, Wf_ref[...].T,
                               preferred_element_type=jnp.float32) + bf_ref[...]


def _stage_final(aggs, hs, Wl3, bl3, Wr3, batch_lanes, W_flat, b_flat):
    return pl.pallas_call(
        lambda *refs: _t3_body(refs),
        grid=(N_BLOCKS,),
        in_specs=(
            [pl.BlockSpec((PB, 128), lambda i: (i, 0))] * 8
            + [pl.BlockSpec((FEAT, 64), lambda i: (0, 0)),
               pl.BlockSpec((1, FEAT), lambda i: (0, 0)),
               pl.BlockSpec((FEAT, 64), lambda i: (0, 0)),
               pl.BlockSpec((1, 8, PB), lambda i: (i, 0, 0)),
               pl.BlockSpec((256, FEAT), lambda i: (0, 0)),
               pl.BlockSpec((1, 256), lambda i: (0, 0))]
        ),
        out_specs=pl.BlockSpec((N_GRAPHS, 256), lambda i: (0, 0)),
        out_shape=jax.ShapeDtypeStruct((N_GRAPHS, 256), jnp.float32),
        scratch_shapes=[pltpu.VMEM((N_GRAPHS, FEAT + 8), jnp.float32)],
    )(*aggs, *hs, Wl3, bl3.reshape(1, FEAT), Wr3, batch_lanes,
      W_flat, b_flat.reshape(1, 256))


def kernel(x, edge_index, batch, W_emb, b_emb, Wl1, bl1, Wr1, Wl2, bl2, Wr2,
           Wl3, bl3, Wr3, W_flat, b_flat):
    src = edge_index[0].reshape(-1, CHUNK)
    dst = edge_index[1].reshape(-1, CHUNK)
    xp = x.reshape(-1, 8 * FEAT)

    batch_pad = jnp.concatenate(
        [batch, jnp.full((NP - N_NODES,), N_GRAPHS, batch.dtype)])
    batch_lanes = batch_pad.reshape(N_BLOCKS, PB, 8).transpose(0, 2, 1)

    G = jnp.stack([_kron8(W_emb[s * SLAB:(s + 1) * SLAB, :].T)
                   for s in range(2)])
    bpk0 = _pack_bias(b_emb)
    BDL1, BDR1, bpk1 = _bd_weights(Wl1), _bd_weights(Wr1), _pack_bias(bl1)
    BDL2, BDR2, bpk2 = _bd_weights(Wl2), _bd_weights(Wr2), _pack_bias(bl2)

    h0 = _stage_emb(xp, G, bpk0)
    a1 = _seg(_seg_sum_2, src, dst, h0)
    h1 = _stage_mix(a1, h0, BDL1, BDR1, bpk1, relu=True)
    a2 = _seg(_seg_sum_2, src, dst, h1)
    h2 = _stage_mix(a2, h1, BDL2, BDR2, bpk2, relu=True)
    a3 = _seg(_seg_sum_4, src, dst, h2)
    return _stage_final(a3, h2, Wl3, bl3, Wr3, batch_lanes, W_flat, b_flat)
